# baseline (speedup 1.0000x reference)
import jax
import jax.numpy as jnp
from jax.experimental import pallas as pl
from jax.experimental.pallas import tpu as pltpu


def _dma_gather_kernel(ids_ref, head_ref, table_ref, out_ref, gbuf, sem):
    i = pl.program_id(0)
    tb = head_ref.shape[0]
    base = i * tb
    for r in range(tb):
        idx = ids_ref[base + r]
        pltpu.make_async_copy(table_ref.at[pl.ds(idx, 1)],
                              gbuf.at[pl.ds(r, 1)], sem).start()
    pltpu.make_async_copy(table_ref.at[pl.ds(0, tb)],
                          gbuf.at[pl.ds(0, tb)], sem).wait()
    out_ref[...] = head_ref[...] + gbuf[...]


def kernel(head_embed, rel_ids, embed_table):
    B, D = head_embed.shape
    R, _ = embed_table.shape
    tb = 512
    grid_b = pl.cdiv(B, tb)
    ids_1d = rel_ids.astype(jnp.int32).reshape(B)

    return pl.pallas_call(
        _dma_gather_kernel,
        out_shape=jax.ShapeDtypeStruct((B, D), head_embed.dtype),
        grid_spec=pltpu.PrefetchScalarGridSpec(
            num_scalar_prefetch=1,
            grid=(grid_b,),
            in_specs=[
                pl.BlockSpec((tb, D), lambda i, ids: (i, 0)),
                pl.BlockSpec((R, D), lambda i, ids: (0, 0)),
            ],
            out_specs=pl.BlockSpec((tb, D), lambda i, ids: (i, 0)),
            scratch_shapes=[
                pltpu.VMEM((tb, D), embed_table.dtype),
                pltpu.SemaphoreType.DMA(()),
            ],
        ),
        compiler_params=pltpu.CompilerParams(
            dimension_semantics=("parallel",),
        ),
    )(ids_1d, head_embed, embed_table)


# speedup vs baseline: 3.9387x; 3.9387x over previous
import jax
import jax.numpy as jnp
from jax.experimental import pallas as pl
from jax.experimental.pallas import tpu as pltpu

_MM_FRAC_NUM = 5
_MM_FRAC_DEN = 8


def _hybrid_kernel(cbase_ref, shift_ref, ids_ref, head_ref, table_f32_ref,
                   table_bf16_ref, out_ref):
    i = pl.program_id(0)
    tb, D = head_ref.shape
    R = table_f32_ref.shape[0]
    mm = (tb * _MM_FRAC_NUM // _MM_FRAC_DEN) & ~7
    base = i * tb

    ids_mm = ids_ref[pl.ds(0, mm), :]
    iota_r = jax.lax.broadcasted_iota(jnp.int32, (mm, R), 1)
    one_hot = (iota_r == ids_mm).astype(jnp.bfloat16)
    gathered_mm = jnp.dot(one_hot, table_bf16_ref[...],
                          preferred_element_type=jnp.float32)
    out_ref[pl.ds(0, mm), :] = head_ref[pl.ds(0, mm), :] + gathered_mm

    iota8 = jax.lax.broadcasted_iota(jnp.int32, (8, D), 0)
    masks = [(iota8 == r).astype(jnp.float32) for r in range(8)]
    for c in range((tb - mm) // 8):
        row0 = mm + c * 8
        parts = []
        for r in range(8):
            b = base + row0 + r
            chunk_base = pl.multiple_of(cbase_ref[b], 8)
            chunk = table_f32_ref[pl.ds(chunk_base, 8), :]
            rolled = pltpu.roll(chunk, shift_ref[b], axis=0)
            parts.append(rolled * masks[r])
        g01 = parts[0] + parts[1]
        g23 = parts[2] + parts[3]
        g45 = parts[4] + parts[5]
        g67 = parts[6] + parts[7]
        gathered = (g01 + g23) + (g45 + g67)
        out_ref[pl.ds(row0, 8), :] = head_ref[pl.ds(row0, 8), :] + gathered


def kernel(head_embed, rel_ids, embed_table):
    B, D = head_embed.shape
    R, _ = embed_table.shape
    tb = max(t for t in (2048, 1024, 512, 256, 128, 64, 32, 16, 8)
             if B % t == 0 or t == 8)
    grid_b = pl.cdiv(B, tb)

    ids_1d = rel_ids.astype(jnp.int32).reshape(B)
    ids_2d = ids_1d.reshape(B, 1)
    cbase_1d = (ids_1d >> 3) << 3
    shift_1d = ((jnp.arange(B, dtype=jnp.int32) & 7) - (ids_1d & 7)) & 7
    table_bf16 = embed_table.astype(jnp.bfloat16)

    return pl.pallas_call(
        _hybrid_kernel,
        out_shape=jax.ShapeDtypeStruct((B, D), head_embed.dtype),
        grid_spec=pltpu.PrefetchScalarGridSpec(
            num_scalar_prefetch=2,
            grid=(grid_b,),
            in_specs=[
                pl.BlockSpec((tb, 1), lambda i, cb, sh: (i, 0)),
                pl.BlockSpec((tb, D), lambda i, cb, sh: (i, 0)),
                pl.BlockSpec((R, D), lambda i, cb, sh: (0, 0)),
                pl.BlockSpec((R, D), lambda i, cb, sh: (0, 0)),
            ],
            out_specs=pl.BlockSpec((tb, D), lambda i, cb, sh: (i, 0)),
        ),
        compiler_params=pltpu.CompilerParams(
            dimension_semantics=("parallel",),
        ),
    )(cbase_1d, shift_1d, ids_2d, head_embed, embed_table, table_bf16)


# speedup vs baseline: 4.2462x; 1.0781x over previous
import jax
import jax.numpy as jnp
from jax.experimental import pallas as pl
from jax.experimental.pallas import tpu as pltpu

_MM_CHUNKS = 8


def _hybrid_kernel(packed_ref, ids_ref, head_ref, table_f32_ref,
                   table_bf16_ref, out_ref):
    i = pl.program_id(0)
    tb, D = head_ref.shape
    R = table_f32_ref.shape[0]
    mm = (tb * 3 // 4) & ~(8 * _MM_CHUNKS - 1)
    mc = mm // _MM_CHUNKS
    n_groups = (tb - mm) // 8
    base = i * tb

    iota_mc = jax.lax.broadcasted_iota(jnp.int32, (mc, R), 1)
    iota8 = jax.lax.broadcasted_iota(jnp.int32, (8, D), 0)
    masks = [(iota8 == r).astype(jnp.float32) for r in range(8)]
    table_bf16 = table_bf16_ref[...]

    def gather_group(c):
        row0 = mm + c * 8
        parts = []
        for r in range(8):
            v = packed_ref[base + row0 + r]
            chunk_base = pl.multiple_of(v & ((R - 1) & ~7), 8)
            chunk = table_f32_ref[pl.ds(chunk_base, 8), :]
            rolled = pltpu.roll(chunk, v >> 12, axis=0)
            parts.append(rolled * masks[r])
        g01 = parts[0] + parts[1]
        g23 = parts[2] + parts[3]
        g45 = parts[4] + parts[5]
        g67 = parts[6] + parts[7]
        gathered = (g01 + g23) + (g45 + g67)
        out_ref[pl.ds(row0, 8), :] = head_ref[pl.ds(row0, 8), :] + gathered

    if mc == 0:
        for c in range(n_groups):
            gather_group(c)
        return

    g_next = 0
    for k in range(_MM_CHUNKS):
        r0 = k * mc
        ids_k = ids_ref[pl.ds(r0, mc), :]
        one_hot = (iota_mc == ids_k).astype(jnp.bfloat16)
        dot_k = jnp.dot(one_hot, table_bf16,
                        preferred_element_type=jnp.float32)
        g_end = (k + 1) * n_groups // _MM_CHUNKS
        while g_next < g_end:
            gather_group(g_next)
            g_next += 1
        out_ref[pl.ds(r0, mc), :] = head_ref[pl.ds(r0, mc), :] + dot_k


def kernel(head_embed, rel_ids, embed_table):
    B, D = head_embed.shape
    R, _ = embed_table.shape
    tb = max(t for t in (2048, 1024, 512, 256, 128, 64, 32, 16, 8)
             if B % t == 0 or t == 8)
    grid_b = pl.cdiv(B, tb)

    ids_1d = rel_ids.astype(jnp.int32).reshape(B)
    ids_2d = ids_1d.reshape(B, 1)
    shift_1d = ((jnp.arange(B, dtype=jnp.int32) & 7) - (ids_1d & 7)) & 7
    packed_1d = ((ids_1d >> 3) << 3) | (shift_1d << 12)
    table_bf16 = embed_table.astype(jnp.bfloat16)

    return pl.pallas_call(
        _hybrid_kernel,
        out_shape=jax.ShapeDtypeStruct((B, D), head_embed.dtype),
        grid_spec=pltpu.PrefetchScalarGridSpec(
            num_scalar_prefetch=1,
            grid=(grid_b,),
            in_specs=[
                pl.BlockSpec((tb, 1), lambda i, pk: (i, 0)),
                pl.BlockSpec((tb, D), lambda i, pk: (i, 0)),
                pl.BlockSpec((R, D), lambda i, pk: (0, 0)),
                pl.BlockSpec((R, D), lambda i, pk: (0, 0)),
            ],
            out_specs=pl.BlockSpec((tb, D), lambda i, pk: (i, 0)),
        ),
        compiler_params=pltpu.CompilerParams(
            dimension_semantics=("parallel",),
        ),
    )(packed_1d, ids_2d, head_embed, embed_table, table_bf16)


# speedup vs baseline: 5.3699x; 1.2646x over previous
import jax
import jax.numpy as jnp
from jax.experimental import pallas as pl
from jax.experimental.pallas import tpu as pltpu

_CHUNKS = 8


def _onehot_kernel(ids_ref, head_ref, table_ref, out_ref):
    tb, D = head_ref.shape
    R = table_ref.shape[0]
    mc = tb // _CHUNKS
    iota_mc = jax.lax.broadcasted_iota(jnp.int32, (mc, R), 1)
    table = table_ref[...]
    dots = []
    for k in range(_CHUNKS):
        ids_k = ids_ref[pl.ds(k * mc, mc), :]
        one_hot = (iota_mc == ids_k).astype(jnp.bfloat16)
        dots.append(jnp.dot(one_hot, table, preferred_element_type=jnp.float32))
        if k >= 1:
            r0 = (k - 1) * mc
            out_ref[pl.ds(r0, mc), :] = head_ref[pl.ds(r0, mc), :] + dots[k - 1]
    r0 = (_CHUNKS - 1) * mc
    out_ref[pl.ds(r0, mc), :] = head_ref[pl.ds(r0, mc), :] + dots[-1]


def kernel(head_embed, rel_ids, embed_table):
    B, D = head_embed.shape
    R, _ = embed_table.shape
    tb = 2048
    grid_b = pl.cdiv(B, tb)
    ids_2d = rel_ids.astype(jnp.int32).reshape(B, 1)
    table_bf16 = embed_table.astype(jnp.bfloat16)

    return pl.pallas_call(
        _onehot_kernel,
        out_shape=jax.ShapeDtypeStruct((B, D), head_embed.dtype),
        grid=(grid_b,),
        in_specs=[
            pl.BlockSpec((tb, 1), lambda i: (i, 0)),
            pl.BlockSpec((tb, D), lambda i: (i, 0)),
            pl.BlockSpec((R, D), lambda i: (0, 0)),
        ],
        out_specs=pl.BlockSpec((tb, D), lambda i: (i, 0)),
        compiler_params=pltpu.CompilerParams(
            dimension_semantics=("parallel",),
        ),
    )(ids_2d, head_embed, table_bf16)
